# Initial kernel scaffold; baseline (speedup 1.0000x reference)
import jax
import jax.numpy as jnp
from jax.experimental import pallas as pl


def kernel(x, edge_index, W0, b0, W1, b1, We1, be1, We2, be2):
    raise NotImplementedError("write your pallas kernel here")


# speedup vs baseline: 4.8011x; 4.8011x over previous
import functools

import jax
import jax.numpy as jnp
from jax import lax
from jax.experimental import pallas as pl
from jax.experimental.pallas import tpu as pltpu
from jax.experimental.pallas import tpu_sc as plsc

_NC = 2
_NS = 16
_B = 80


def _sc_segment_sum(table, src3, dst3, n_rows):
    kc = src3.shape[1]
    d = table.shape[1]
    rps = n_rows // _NS

    @functools.partial(
        pl.kernel,
        out_type=jax.ShapeDtypeStruct((_NC, n_rows, d), jnp.float32),
        mesh=plsc.VectorSubcoreMesh(core_axis_name="c", subcore_axis_name="s"),
        compiler_params=pltpu.CompilerParams(has_side_effects=True),
        scratch_types=[
            pltpu.VMEM((kc, _B), jnp.int32),
            pltpu.VMEM((kc, _B), jnp.int32),
            pltpu.VMEM((_B, d), jnp.float32),
            pltpu.VMEM_SHARED((n_rows, d), jnp.float32),
            pltpu.SemaphoreType.DMA,
        ],
    )
    def k(table_h, src_h, dst_h, zeros_h, out_h, src_v, dst_v, rows_v, acc, sem):
        c = lax.axis_index("c")
        s = lax.axis_index("s")
        wid = c * _NS + s
        pltpu.sync_copy(zeros_h, acc.at[pl.ds(s * rps, rps)])
        pltpu.sync_copy(src_h.at[wid], src_v)
        pltpu.sync_copy(dst_h.at[wid], dst_v)
        plsc.subcore_barrier()

        def body(j, carry):
            pltpu.async_copy(table_h.at[src_v.at[j]], rows_v, sem).wait()
            pltpu.sync_copy(rows_v, acc.at[dst_v.at[j]], add=True)
            return carry

        lax.fori_loop(0, kc, body, 0)
        plsc.subcore_barrier()
        pltpu.sync_copy(acc.at[pl.ds(s * rps, rps)],
                        out_h.at[c, pl.ds(s * rps, rps)])

    return k(table, src3, dst3, jnp.zeros((rps, d), jnp.float32))


def _sc_gather_pairs(table, src3, dst3, e):
    kc = src3.shape[1]
    d = table.shape[1]
    ew = e // (_NC * _NS)

    @functools.partial(
        pl.kernel,
        out_type=[jax.ShapeDtypeStruct((e, d), jnp.float32),
                  jax.ShapeDtypeStruct((e, d), jnp.float32)],
        mesh=plsc.VectorSubcoreMesh(core_axis_name="c", subcore_axis_name="s"),
        compiler_params=pltpu.CompilerParams(has_side_effects=True),
        scratch_types=[
            pltpu.VMEM((kc, _B), jnp.int32),
            pltpu.VMEM((kc, _B), jnp.int32),
            pltpu.VMEM((_B, d), jnp.float32),
            pltpu.VMEM((_B, d), jnp.float32),
            pltpu.SemaphoreType.DMA,
            pltpu.SemaphoreType.DMA,
        ],
    )
    def k(table_h, src_h, dst_h, hs_h, hd_h, src_v, dst_v, ra, rb, sa, sb):
        c = lax.axis_index("c")
        s = lax.axis_index("s")
        wid = c * _NS + s
        pltpu.sync_copy(src_h.at[wid], src_v)
        pltpu.sync_copy(dst_h.at[wid], dst_v)

        def body(j, carry):
            base = wid * ew + j * _B
            da = pltpu.async_copy(table_h.at[src_v.at[j]], ra, sa)
            db = pltpu.async_copy(table_h.at[dst_v.at[j]], rb, sb)
            da.wait()
            pltpu.sync_copy(ra, hs_h.at[pl.ds(base, _B)])
            db.wait()
            pltpu.sync_copy(rb, hd_h.at[pl.ds(base, _B)])
            return carry

        lax.fori_loop(0, kc, body, 0)

    return k(table, src3, dst3)


def _tc_xw_aug(x, W0, blk=1024):
    n, din = x.shape
    hid = W0.shape[1]
    daug = 2 * hid

    def body(x_ref, w_ref, o_ref):
        xw = jnp.dot(x_ref[...], w_ref[...], preferred_element_type=jnp.float32)
        pad = (lax.broadcasted_iota(jnp.int32, (blk, hid), 1) == 0).astype(jnp.float32)
        o_ref[...] = jnp.concatenate([xw, pad], axis=1)

    return pl.pallas_call(
        body,
        grid=(n // blk,),
        in_specs=[pl.BlockSpec((blk, din), lambda i: (i, 0)),
                  pl.BlockSpec((din, hid), lambda i: (0, 0))],
        out_specs=pl.BlockSpec((blk, daug), lambda i: (i, 0)),
        out_shape=jax.ShapeDtypeStruct((n, daug), jnp.float32),
    )(x, W0)


def _tc_layer1(xw, parts, b0, W1, blk=1024):
    n, daug = xw.shape
    hid = b0.shape[0]

    def body(xw_ref, p_ref, b_ref, w_ref, h1w_ref, dinv_ref):
        p = p_ref[0] + p_ref[1]
        deg = jnp.maximum(p[:, hid:hid + 1], 1.0)
        dinv = 1.0 / deg
        h1 = jnp.maximum(xw_ref[:, :hid] + p[:, :hid] * dinv + b_ref[...], 0.0)
        h1w = jnp.dot(h1, w_ref[...], preferred_element_type=jnp.float32)
        h1w_ref[...] = jnp.concatenate(
            [h1w, jnp.zeros((blk, daug - hid), jnp.float32)], axis=1)
        dinv_ref[...] = dinv

    return pl.pallas_call(
        body,
        grid=(n // blk,),
        in_specs=[pl.BlockSpec((blk, daug), lambda i: (i, 0)),
                  pl.BlockSpec((2, blk, daug), lambda i: (0, i, 0)),
                  pl.BlockSpec((1, hid), lambda i: (0, 0)),
                  pl.BlockSpec((hid, hid), lambda i: (0, 0))],
        out_specs=[pl.BlockSpec((blk, daug), lambda i: (i, 0)),
                   pl.BlockSpec((blk, 1), lambda i: (i, 0))],
        out_shape=[jax.ShapeDtypeStruct((n, daug), jnp.float32),
                   jax.ShapeDtypeStruct((n, 1), jnp.float32)],
    )(xw, parts, b0.reshape(1, -1), W1)


def _tc_layer2(h1w, parts, dinv, b1, blk=1024):
    n, daug = h1w.shape
    hid = b1.shape[0]

    def body(hw_ref, p_ref, di_ref, b_ref, o_ref):
        p = p_ref[0] + p_ref[1]
        h2 = jnp.maximum(
            hw_ref[:, :hid] + p[:, :hid] * di_ref[...] + b_ref[...], 0.0)
        o_ref[...] = jnp.concatenate(
            [h2, jnp.zeros((blk, daug - hid), jnp.float32)], axis=1)

    return pl.pallas_call(
        body,
        grid=(n // blk,),
        in_specs=[pl.BlockSpec((blk, daug), lambda i: (i, 0)),
                  pl.BlockSpec((2, blk, daug), lambda i: (0, i, 0)),
                  pl.BlockSpec((blk, 1), lambda i: (i, 0)),
                  pl.BlockSpec((1, hid), lambda i: (0, 0))],
        out_specs=pl.BlockSpec((blk, daug), lambda i: (i, 0)),
        out_shape=jax.ShapeDtypeStruct((n, daug), jnp.float32),
    )(h1w, parts, dinv, b1.reshape(1, -1))


def _tc_edge_mlp(hs, hd, We1, be1, We2, be2, table, src3, dst3, blk=2000):
    e, dw = hs.shape
    f = We1.shape[0]
    hid = f // 4
    kc = src3.shape[1]

    def body(a_ref, b_ref, w1_ref, b1_ref, w2_ref, b2_ref, t_ref, s_ref, d_ref,
             lg_ref, pr_ref):
        del t_ref, s_ref, d_ref
        a = a_ref[:, :hid]
        b = b_ref[:, :hid]
        ef = jnp.concatenate([a, b, a * b, jnp.abs(a - b)], axis=1)
        h = jnp.maximum(
            jnp.dot(ef, w1_ref[...], preferred_element_type=jnp.float32) + b1_ref[...],
            0.0)
        lg = jnp.dot(h, w2_ref[...], preferred_element_type=jnp.float32) + b2_ref[...]
        lg_ref[...] = lg
        pr_ref[...] = 1.0 / (1.0 + jnp.exp(-lg))

    return pl.pallas_call(
        body,
        grid=(e // blk,),
        in_specs=[pl.BlockSpec((blk, dw), lambda i: (i, 0)),
                  pl.BlockSpec((blk, dw), lambda i: (i, 0)),
                  pl.BlockSpec((f, We1.shape[1]), lambda i: (0, 0)),
                  pl.BlockSpec((1, be1.shape[1]), lambda i: (0, 0)),
                  pl.BlockSpec((We2.shape[0], 1), lambda i: (0, 0)),
                  pl.BlockSpec((1, 1), lambda i: (0, 0)),
                  pl.BlockSpec((8, table.shape[1]), lambda i: (0, 0)),
                  pl.BlockSpec((1, kc, _B), lambda i: (0, 0, 0)),
                  pl.BlockSpec((1, kc, _B), lambda i: (0, 0, 0))],
        out_specs=[pl.BlockSpec((blk, 1), lambda i: (i, 0)),
                   pl.BlockSpec((blk, 1), lambda i: (i, 0))],
        out_shape=[jax.ShapeDtypeStruct((e, 1), jnp.float32),
                   jax.ShapeDtypeStruct((e, 1), jnp.float32)],
    )(hs, hd, We1, be1, We2, be2, table, src3, dst3)


def kernel(x, edge_index, W0, b0, W1, b1, We1, be1, We2, be2):
    n = x.shape[0]
    e = edge_index.shape[1]
    hid = W0.shape[1]
    nw = _NC * _NS
    ew = e // nw
    kc = ew // _B
    src3 = edge_index[0].reshape(nw, kc, _B)
    dst3 = edge_index[1].reshape(nw, kc, _B)

    np_ = ((n + 8 * _NS - 1) // (8 * _NS)) * (8 * _NS)
    x_pad = jnp.pad(x, ((0, np_ - n), (0, 0)))

    xwa = _tc_xw_aug(x_pad, W0)
    parts1 = _sc_segment_sum(xwa, src3, dst3, np_)
    h1w, dinv = _tc_layer1(xwa, parts1, b0, W1)
    parts2 = _sc_segment_sum(h1w, src3, dst3, np_)
    Hp = _tc_layer2(h1w, parts2, dinv, b1)
    hs, hd = _sc_gather_pairs(Hp, src3, dst3, e)
    lg, pr = _tc_edge_mlp(hs, hd, We1, be1.reshape(1, -1),
                          We2, be2.reshape(1, 1), Hp, src3, dst3)
    return Hp[:n, :hid], lg[:, 0], pr[:, 0]
